# Initial kernel scaffold; baseline (speedup 1.0000x reference)
import jax
import jax.numpy as jnp
from jax.experimental import pallas as pl


def kernel(uv, m_u, m_v):
    raise NotImplementedError("write your pallas kernel here")


# speedup vs baseline: 4.1454x; 4.1454x over previous
import functools

import jax
import jax.numpy as jnp
from jax import lax
from jax.experimental import pallas as pl
from jax.experimental.pallas import tpu as pltpu
from jax.experimental.pallas import tpu_sc as plsc

_NW = 32
_LANES = 16


def _lpe_body(L, B, nblk_total, uv_hbm, mu_hbm, mv_hbm, out_hbm,
              uv_v, out_v, mu_v, mv_v):
    cid = lax.axis_index("c")
    sid = lax.axis_index("s")
    wid = sid * 2 + cid
    vpb = B // _LANES
    kmax = (nblk_total + _NW - 1) // _NW

    pltpu.sync_copy(mu_hbm, mu_v)
    pltpu.sync_copy(mv_hbm, mv_v)

    iota = lax.iota(jnp.int32, _LANES)
    z16 = jnp.zeros((_LANES,), jnp.int32)
    maxi = jnp.full((_LANES,), L - 1, jnp.int32)
    cL = jnp.full((_LANES,), L, jnp.int32)
    fscale = jnp.float32(L - 1)

    def lerp2(tab_v, i0, i1, w):
        a0 = plsc.load_gather(tab_v, [i0])
        a1 = plsc.load_gather(tab_v, [i1])
        b0 = plsc.load_gather(tab_v, [i0 + cL])
        b1 = plsc.load_gather(tab_v, [i1 + cL])
        return a0 + w * (a1 - a0), b0 + w * (b1 - b0)

    def vec_body(jj, _):
        pt = jj * _LANES + iota
        p2 = pt * 2
        u = plsc.load_gather(uv_v, [p2])
        v = plsc.load_gather(uv_v, [p2 + 1])

        tu = (u + 1.0) * 0.5 * fscale
        tv = (v + 1.0) * 0.5 * fscale
        iu0 = tu.astype(jnp.int32)
        iv0 = tv.astype(jnp.int32)
        wu = tu - iu0.astype(jnp.float32)
        wv = tv - iv0.astype(jnp.float32)
        iu0 = jnp.minimum(jnp.maximum(iu0, z16), maxi)
        iv0 = jnp.minimum(jnp.maximum(iv0, z16), maxi)
        iu1 = jnp.minimum(iu0 + 1, maxi)
        iv1 = jnp.minimum(iv0 + 1, maxi)

        mu0, mu1 = lerp2(mu_v, iu0, iu1, wu)
        mv0, mv1 = lerp2(mv_v, iv0, iv1, wv)

        p4 = pt * 4
        plsc.store_scatter(out_v, [p4], mu0)
        plsc.store_scatter(out_v, [p4 + 1], mu1)
        plsc.store_scatter(out_v, [p4 + 2], mv0)
        plsc.store_scatter(out_v, [p4 + 3], mv1)
        return 0

    def blk_body(k, _):
        b = k * _NW + wid

        @pl.when(b < nblk_total)
        def _():
            base = b * B
            pltpu.sync_copy(uv_hbm.at[pl.ds(base * 2, B * 2)], uv_v)
            lax.fori_loop(0, vpb, vec_body, 0)
            pltpu.sync_copy(out_v, out_hbm.at[pl.ds(base * 4, B * 4)])

        return 0

    lax.fori_loop(0, kmax, blk_body, 0)


def kernel(uv, m_u, m_v):
    N = uv.shape[0]
    L = m_u.shape[1]
    B = 2000
    nblk_total = N // B

    mesh = plsc.VectorSubcoreMesh(core_axis_name="c", subcore_axis_name="s")
    f = pl.kernel(
        functools.partial(_lpe_body, L, B, nblk_total),
        out_type=jax.ShapeDtypeStruct((N * 4,), jnp.float32),
        mesh=mesh,
        compiler_params=pltpu.CompilerParams(needs_layout_passes=False),
        scratch_types=[
            pltpu.VMEM((B * 2,), jnp.float32),
            pltpu.VMEM((B * 4,), jnp.float32),
            pltpu.VMEM((2 * L,), jnp.float32),
            pltpu.VMEM((2 * L,), jnp.float32),
        ],
    )
    out_flat = f(uv.reshape(N * 2), m_u.reshape(2 * L), m_v.reshape(2 * L))
    return out_flat.reshape(N, 4)


# speedup vs baseline: 4.2016x; 1.0135x over previous
import functools

import jax
import jax.numpy as jnp
from jax import lax
from jax.experimental import pallas as pl
from jax.experimental.pallas import tpu as pltpu
from jax.experimental.pallas import tpu_sc as plsc

_NW = 32
_LANES = 16


def _lpe_body(L, B, nblk_total, uv_hbm, mu_hbm, mv_hbm, out_hbm,
              uv_v, out_v, mu_v, mv_v):
    cid = lax.axis_index("c")
    sid = lax.axis_index("s")
    wid = sid * 2 + cid
    vpb = B // _LANES
    kmax = (nblk_total + _NW - 1) // _NW

    pltpu.sync_copy(mu_hbm, mu_v)
    pltpu.sync_copy(mv_hbm, mv_v)

    iota = lax.iota(jnp.int32, _LANES)
    z16 = jnp.zeros((_LANES,), jnp.int32)
    maxi = jnp.full((_LANES,), L - 1, jnp.int32)
    cL = jnp.full((_LANES,), L, jnp.int32)
    fscale = jnp.float32(L - 1)

    def lerp2(tab_v, i0, i1, w):
        a0 = plsc.load_gather(tab_v, [i0])
        a1 = plsc.load_gather(tab_v, [i1])
        b0 = plsc.load_gather(tab_v, [i0 + cL])
        b1 = plsc.load_gather(tab_v, [i1 + cL])
        return a0 + w * (a1 - a0), b0 + w * (b1 - b0)

    def vec_body(jj):
        pt = jj * _LANES + iota
        p2 = pt * 2
        u = plsc.load_gather(uv_v, [p2])
        v = plsc.load_gather(uv_v, [p2 + 1])

        tu = (u + 1.0) * 0.5 * fscale
        tv = (v + 1.0) * 0.5 * fscale
        iu0 = tu.astype(jnp.int32)
        iv0 = tv.astype(jnp.int32)
        wu = tu - iu0.astype(jnp.float32)
        wv = tv - iv0.astype(jnp.float32)
        iu0 = jnp.minimum(jnp.maximum(iu0, z16), maxi)
        iv0 = jnp.minimum(jnp.maximum(iv0, z16), maxi)
        iu1 = jnp.minimum(iu0 + 1, maxi)
        iv1 = jnp.minimum(iv0 + 1, maxi)

        mu0, mu1 = lerp2(mu_v, iu0, iu1, wu)
        mv0, mv1 = lerp2(mv_v, iv0, iv1, wv)

        p4 = pt * 4
        plsc.store_scatter(out_v, [p4], mu0)
        plsc.store_scatter(out_v, [p4 + 1], mu1)
        plsc.store_scatter(out_v, [p4 + 2], mv0)
        plsc.store_scatter(out_v, [p4 + 3], mv1)

    def blk_body(k, _):
        b = k * _NW + wid

        @pl.when(b < nblk_total)
        def _():
            base = b * B
            pltpu.sync_copy(uv_hbm.at[pl.ds(base * 2, B * 2)], uv_v)
            plsc.parallel_loop(0, vpb, unroll=8)(vec_body)
            pltpu.sync_copy(out_v, out_hbm.at[pl.ds(base * 4, B * 4)])

        return 0

    lax.fori_loop(0, kmax, blk_body, 0)


def kernel(uv, m_u, m_v):
    N = uv.shape[0]
    L = m_u.shape[1]
    B = 2000
    nblk_total = N // B

    mesh = plsc.VectorSubcoreMesh(core_axis_name="c", subcore_axis_name="s")
    f = pl.kernel(
        functools.partial(_lpe_body, L, B, nblk_total),
        out_type=jax.ShapeDtypeStruct((N * 4,), jnp.float32),
        mesh=mesh,
        compiler_params=pltpu.CompilerParams(needs_layout_passes=False),
        scratch_types=[
            pltpu.VMEM((B * 2,), jnp.float32),
            pltpu.VMEM((B * 4,), jnp.float32),
            pltpu.VMEM((2 * L,), jnp.float32),
            pltpu.VMEM((2 * L,), jnp.float32),
        ],
    )
    out_flat = f(uv.reshape(N * 2), m_u.reshape(2 * L), m_v.reshape(2 * L))
    return out_flat.reshape(N, 4)
